# Initial kernel scaffold; baseline (speedup 1.0000x reference)
import jax
import jax.numpy as jnp
from jax.experimental import pallas as pl


def kernel(k_cache, v_cache, latest_k, latest_v, position_ids):
    raise NotImplementedError("write your pallas kernel here")


# speedup vs baseline: 4.1592x; 4.1592x over previous
import jax
import jax.numpy as jnp
from jax.experimental import pallas as pl
from jax.experimental.pallas import tpu as pltpu

L, B, H, S, D = 2, 8, 4, 2048, 128


def _body(pos_ref, k_ref, v_ref, lk_ref, lv_ref, out_ref):
    b = pl.program_id(1)
    pos = pos_ref[b]
    out_ref[0, 0, 0, 0] = k_ref[0, 0, 0]
    out_ref[1, 0, 0, 0] = v_ref[0, 0, 0]
    out_ref[0, 0, 0, 0, pl.ds(pos, 1), :] = lk_ref[0, 0, 0]
    out_ref[1, 0, 0, 0, pl.ds(pos, 1), :] = lv_ref[0, 0, 0]


def kernel(k_cache, v_cache, latest_k, latest_v, position_ids):
    grid_spec = pltpu.PrefetchScalarGridSpec(
        num_scalar_prefetch=1,
        grid=(L, B, H),
        in_specs=[
            pl.BlockSpec((1, 1, 1, S, D), lambda l, b, h, pos: (l, b, h, 0, 0)),
            pl.BlockSpec((1, 1, 1, S, D), lambda l, b, h, pos: (l, b, h, 0, 0)),
            pl.BlockSpec((1, 1, 1, 1, D), lambda l, b, h, pos: (l, b, h, 0, 0)),
            pl.BlockSpec((1, 1, 1, 1, D), lambda l, b, h, pos: (l, b, h, 0, 0)),
        ],
        out_specs=pl.BlockSpec(
            (2, 1, 1, 1, S, D), lambda l, b, h, pos: (0, l, b, h, 0, 0)
        ),
    )
    return pl.pallas_call(
        _body,
        grid_spec=grid_spec,
        out_shape=jax.ShapeDtypeStruct((2, L, B, H, S, D), jnp.float32),
        compiler_params=pltpu.CompilerParams(
            dimension_semantics=("arbitrary", "arbitrary", "arbitrary"),
        ),
    )(position_ids[:, 0], k_cache, v_cache, latest_k, latest_v)
